# baseline (speedup 1.0000x reference)
import functools

import jax
import jax.numpy as jnp
from jax import lax
from jax.experimental import pallas as pl
from jax.experimental.pallas import tpu as pltpu
from jax.experimental.pallas import tpu_sc as plsc

_C = 200
_NW = 32


def _mlp_body(x_ref, w1_ref, b1_ref, w2_ref, b2_ref, w3_ref, b3_ref, out_ref):
    h = jnp.dot(x_ref[...], w1_ref[...], preferred_element_type=jnp.float32)
    h = jnp.maximum(h + b1_ref[...], 0.0)
    d = jnp.dot(h, w2_ref[...], preferred_element_type=jnp.float32) + b2_ref[...]
    out_ref[...] = (
        jnp.dot(d, w3_ref[...], preferred_element_type=jnp.float32) + b3_ref[...]
    )


def _mlp(x, W1, b1, W2, b2, W3, b3):
    n, d = x.shape
    h = W1.shape[1]
    h2 = W2.shape[1]
    out_d = W3.shape[1]
    blk = 1000
    return pl.pallas_call(
        _mlp_body,
        grid=(n // blk,),
        in_specs=[
            pl.BlockSpec((blk, d), lambda i: (i, 0)),
            pl.BlockSpec((d, h), lambda i: (0, 0)),
            pl.BlockSpec((1, h), lambda i: (0, 0)),
            pl.BlockSpec((h, h2), lambda i: (0, 0)),
            pl.BlockSpec((1, h2), lambda i: (0, 0)),
            pl.BlockSpec((h2, out_d), lambda i: (0, 0)),
            pl.BlockSpec((1, out_d), lambda i: (0, 0)),
        ],
        out_specs=pl.BlockSpec((blk, out_d), lambda i: (i, 0)),
        out_shape=jax.ShapeDtypeStruct((n, out_d), jnp.float32),
    )(x, W1, b1.reshape(1, -1), W2, b2.reshape(1, -1), W3, b3.reshape(1, -1))


_NB = 4


def _sc_gather(x_dnn, x, idx_l, idx_r, n_chunks):
    n_pad, c = idx_l.shape
    d = x.shape[1]
    ipw = n_pad // _NW
    mesh = plsc.VectorSubcoreMesh(core_axis_name="c", subcore_axis_name="s")

    @functools.partial(
        pl.kernel,
        mesh=mesh,
        out_type=[
            jax.ShapeDtypeStruct((2, n_chunks, c, d), jnp.float32),
            jax.ShapeDtypeStruct((2, n_chunks, c, d), jnp.float32),
        ],
        scratch_types=(
            [pltpu.VMEM((_NB, c, d), jnp.float32)]
            + [pltpu.VMEM((c,), jnp.int32) for _ in range(_NB)]
            + [pltpu.SemaphoreType.DMA for _ in range(3 * _NB)]
        ),
    )
    def k(dnn_hbm, x_hbm, idxl_hbm, idxr_hbm, emb_hbm, feat_hbm,
          bufs, *rest):
        gidx = rest[:_NB]
        isems = rest[_NB:2 * _NB]
        gsems = rest[2 * _NB:3 * _NB]
        wsems = rest[3 * _NB:]
        wid = lax.axis_index("s") * 2 + lax.axis_index("c")

        tables = (dnn_hbm, dnn_hbm, x_hbm, x_hbm)
        idx_hbms = (idxl_hbm, idxr_hbm, idxl_hbm, idxr_hbm)
        outs = (emb_hbm, emb_hbm, feat_hbm, feat_hbm)
        sides = (0, 1, 0, 1)
        T = 4 * ipw

        def chunk_of(t):
            return wid + (t % ipw) * _NW

        def always_live(t):
            return (t % ipw) * _NW + _NW - 1 < n_chunks

        def i_copy(t, b):
            j = t // ipw
            return pltpu.make_async_copy(
                idx_hbms[j].at[chunk_of(t)], gidx[b], isems[b])

        def g_copy(t, b):
            j = t // ipw
            return pltpu.make_async_copy(
                tables[j].at[gidx[b]], bufs.at[b], gsems[b])

        def w_copy(t, b):
            j = t // ipw
            return pltpu.make_async_copy(
                bufs.at[b], outs[j].at[sides[j], chunk_of(t)], wsems[b])

        def maybe(live_stat, t, fn):
            if live_stat:
                fn()
            else:
                pl.when(chunk_of(t) < n_chunks)(fn)

        for t in range(min(_NB, T)):
            i_copy(t, t).start()
        for t in range(T + _NB - 1):
            b = t % _NB
            if t < T:
                if t >= _NB:
                    sp = t - _NB
                    maybe(always_live(sp), sp, w_copy(sp, b).wait)
                i_copy(t, b).wait()
                g_copy(t, b).start()
            s = t - (_NB - 1)
            if 0 <= s < T:
                bs = s % _NB
                g_copy(s, bs).wait()
                maybe(always_live(s), s, w_copy(s, bs).start)
            if t + 1 < T and t + 1 >= _NB:
                i_copy(t + 1, (t + 1) % _NB).start()
        for s in range(max(T - _NB, 0), T):
            maybe(always_live(s), s, w_copy(s, s % _NB).wait)

    return k(x_dnn, x, idx_l, idx_r)


def kernel(x, edge_index, pair_idxs_left, pair_idxs_right, y, W_lin, b_lin,
           W_gat, a_src, a_dst, b_gat, W1, b1, W2, b2, W3, b3):
    p = pair_idxs_left.shape[0]
    x_dnn = _mlp(x, W1, b1, W2, b2, W3, b3)
    n_chunks = p // _C
    n_pad = ((n_chunks + _NW - 1) // _NW) * _NW
    pad = ((0, n_pad - n_chunks), (0, 0))
    idx_l = jnp.pad(pair_idxs_left.reshape(-1, _C), pad)
    idx_r = jnp.pad(pair_idxs_right.reshape(-1, _C), pad)
    emb, feat = _sc_gather(x_dnn, x, idx_l, idx_r, n_chunks)
    return (emb.reshape(2, p, -1), feat.reshape(2, p, -1), y)


# speedup vs baseline: 3.0592x; 3.0592x over previous
import functools

import jax
import jax.numpy as jnp
from jax import lax
from jax.experimental import pallas as pl
from jax.experimental.pallas import tpu as pltpu
from jax.experimental.pallas import tpu_sc as plsc

_C = 400
_NW = 32


def _mlp_body(x_ref, w1_ref, b1_ref, w2_ref, b2_ref, w3_ref, b3_ref, out_ref):
    h = jnp.dot(x_ref[...], w1_ref[...], preferred_element_type=jnp.float32)
    h = jnp.maximum(h + b1_ref[...], 0.0)
    d = jnp.dot(h, w2_ref[...], preferred_element_type=jnp.float32) + b2_ref[...]
    out_ref[...] = (
        jnp.dot(d, w3_ref[...], preferred_element_type=jnp.float32) + b3_ref[...]
    )


def _mlp(x, W1, b1, W2, b2, W3, b3):
    n, d = x.shape
    h = W1.shape[1]
    h2 = W2.shape[1]
    out_d = W3.shape[1]
    blk = 1000
    return pl.pallas_call(
        _mlp_body,
        grid=(n // blk,),
        in_specs=[
            pl.BlockSpec((blk, d), lambda i: (i, 0)),
            pl.BlockSpec((d, h), lambda i: (0, 0)),
            pl.BlockSpec((1, h), lambda i: (0, 0)),
            pl.BlockSpec((h, h2), lambda i: (0, 0)),
            pl.BlockSpec((1, h2), lambda i: (0, 0)),
            pl.BlockSpec((h2, out_d), lambda i: (0, 0)),
            pl.BlockSpec((1, out_d), lambda i: (0, 0)),
        ],
        out_specs=pl.BlockSpec((blk, out_d), lambda i: (i, 0)),
        out_shape=jax.ShapeDtypeStruct((n, out_d), jnp.float32),
    )(x, W1, b1.reshape(1, -1), W2, b2.reshape(1, -1), W3, b3.reshape(1, -1))


_NB = 4


def _sc_gather(x_dnn, x, idx_l, idx_r, n_chunks):
    n_pad, c = idx_l.shape
    d = x.shape[1]
    ipw = n_pad // _NW
    mesh = plsc.VectorSubcoreMesh(core_axis_name="c", subcore_axis_name="s")

    @functools.partial(
        pl.kernel,
        mesh=mesh,
        out_type=[
            jax.ShapeDtypeStruct((2, n_chunks, c, d), jnp.float32),
            jax.ShapeDtypeStruct((2, n_chunks, c, d), jnp.float32),
        ],
        scratch_types=[
            pltpu.VMEM((c,), jnp.int32),
            pltpu.VMEM((c, d), jnp.float32),
            pltpu.SemaphoreType.DMA,
        ],
    )
    def k(dnn_hbm, x_hbm, idxl_hbm, idxr_hbm, emb_hbm, feat_hbm,
          idx_v, rows_v, sem):
        wid = lax.axis_index("s") * 2 + lax.axis_index("c")

        for table, idx_hbm, out_hbm, side in (
            (dnn_hbm, idxl_hbm, emb_hbm, 0),
            (dnn_hbm, idxr_hbm, emb_hbm, 1),
            (x_hbm, idxl_hbm, feat_hbm, 0),
            (x_hbm, idxr_hbm, feat_hbm, 1),
        ):
            for i in range(ipw):
                ch = wid + i * _NW
                live = i * _NW + _NW - 1 < n_chunks

                def step(table=table, idx_hbm=idx_hbm, out_hbm=out_hbm,
                         side=side, ch=ch):
                    pltpu.sync_copy(idx_hbm.at[ch], idx_v)
                    pltpu.async_copy(table.at[idx_v], rows_v, sem).wait()
                    pltpu.sync_copy(rows_v, out_hbm.at[side, ch])

                if live:
                    step()
                else:
                    pl.when(ch < n_chunks)(step)

    return k(x_dnn, x, idx_l, idx_r)


def kernel(x, edge_index, pair_idxs_left, pair_idxs_right, y, W_lin, b_lin,
           W_gat, a_src, a_dst, b_gat, W1, b1, W2, b2, W3, b3):
    p = pair_idxs_left.shape[0]
    x_dnn = _mlp(x, W1, b1, W2, b2, W3, b3)
    n_chunks = p // _C
    n_pad = ((n_chunks + _NW - 1) // _NW) * _NW
    pad = ((0, n_pad - n_chunks), (0, 0))
    idx_l = jnp.pad(pair_idxs_left.reshape(-1, _C), pad)
    idx_r = jnp.pad(pair_idxs_right.reshape(-1, _C), pad)
    emb, feat = _sc_gather(x_dnn, x, idx_l, idx_r, n_chunks)
    return (emb.reshape(2, p, -1), feat.reshape(2, p, -1), y)


# speedup vs baseline: 3.4651x; 1.1327x over previous
import functools

import jax
import jax.numpy as jnp
from jax import lax
from jax.experimental import pallas as pl
from jax.experimental.pallas import tpu as pltpu
from jax.experimental.pallas import tpu_sc as plsc

_C = 800
_NW = 32


def _mlp_body(x_ref, w1_ref, b1_ref, w2_ref, b2_ref, w3_ref, b3_ref, out_ref):
    h = jnp.dot(x_ref[...], w1_ref[...], preferred_element_type=jnp.float32)
    h = jnp.maximum(h + b1_ref[...], 0.0)
    d = jnp.dot(h, w2_ref[...], preferred_element_type=jnp.float32) + b2_ref[...]
    out_ref[...] = (
        jnp.dot(d, w3_ref[...], preferred_element_type=jnp.float32) + b3_ref[...]
    )


def _mlp(x, W1, b1, W2, b2, W3, b3):
    n, d = x.shape
    h = W1.shape[1]
    h2 = W2.shape[1]
    out_d = W3.shape[1]
    blk = 1000
    return pl.pallas_call(
        _mlp_body,
        grid=(n // blk,),
        in_specs=[
            pl.BlockSpec((blk, d), lambda i: (i, 0)),
            pl.BlockSpec((d, h), lambda i: (0, 0)),
            pl.BlockSpec((1, h), lambda i: (0, 0)),
            pl.BlockSpec((h, h2), lambda i: (0, 0)),
            pl.BlockSpec((1, h2), lambda i: (0, 0)),
            pl.BlockSpec((h2, out_d), lambda i: (0, 0)),
            pl.BlockSpec((1, out_d), lambda i: (0, 0)),
        ],
        out_specs=pl.BlockSpec((blk, out_d), lambda i: (i, 0)),
        out_shape=jax.ShapeDtypeStruct((n, out_d), jnp.float32),
    )(x, W1, b1.reshape(1, -1), W2, b2.reshape(1, -1), W3, b3.reshape(1, -1))


_NB = 4


def _sc_gather(x_dnn, x, idx_l, idx_r, n_chunks):
    n_pad, c = idx_l.shape
    d = x.shape[1]
    ipw = n_pad // _NW
    mesh = plsc.VectorSubcoreMesh(core_axis_name="c", subcore_axis_name="s")

    @functools.partial(
        pl.kernel,
        mesh=mesh,
        out_type=[
            jax.ShapeDtypeStruct((2, n_chunks, c, d), jnp.float32),
            jax.ShapeDtypeStruct((2, n_chunks, c, d), jnp.float32),
        ],
        scratch_types=[
            pltpu.VMEM((c,), jnp.int32),
            pltpu.VMEM((c, d), jnp.float32),
            pltpu.SemaphoreType.DMA,
        ],
    )
    def k(dnn_hbm, x_hbm, idxl_hbm, idxr_hbm, emb_hbm, feat_hbm,
          idx_v, rows_v, sem):
        wid = lax.axis_index("s") * 2 + lax.axis_index("c")

        for table, idx_hbm, out_hbm, side in (
            (dnn_hbm, idxl_hbm, emb_hbm, 0),
            (dnn_hbm, idxr_hbm, emb_hbm, 1),
            (x_hbm, idxl_hbm, feat_hbm, 0),
            (x_hbm, idxr_hbm, feat_hbm, 1),
        ):
            for i in range(ipw):
                ch = wid + i * _NW
                live = i * _NW + _NW - 1 < n_chunks

                def step(table=table, idx_hbm=idx_hbm, out_hbm=out_hbm,
                         side=side, ch=ch):
                    pltpu.sync_copy(idx_hbm.at[ch], idx_v)
                    pltpu.async_copy(table.at[idx_v], rows_v, sem).wait()
                    pltpu.sync_copy(rows_v, out_hbm.at[side, ch])

                if live:
                    step()
                else:
                    pl.when(ch < n_chunks)(step)

    return k(x_dnn, x, idx_l, idx_r)


def kernel(x, edge_index, pair_idxs_left, pair_idxs_right, y, W_lin, b_lin,
           W_gat, a_src, a_dst, b_gat, W1, b1, W2, b2, W3, b3):
    p = pair_idxs_left.shape[0]
    x_dnn = _mlp(x, W1, b1, W2, b2, W3, b3)
    n_chunks = p // _C
    n_pad = ((n_chunks + _NW - 1) // _NW) * _NW
    pad = ((0, n_pad - n_chunks), (0, 0))
    idx_l = jnp.pad(pair_idxs_left.reshape(-1, _C), pad)
    idx_r = jnp.pad(pair_idxs_right.reshape(-1, _C), pad)
    emb, feat = _sc_gather(x_dnn, x, idx_l, idx_r, n_chunks)
    return (emb.reshape(2, p, -1), feat.reshape(2, p, -1), y)


# speedup vs baseline: 3.4743x; 1.0027x over previous
import functools

import jax
import jax.numpy as jnp
from jax import lax
from jax.experimental import pallas as pl
from jax.experimental.pallas import tpu as pltpu
from jax.experimental.pallas import tpu_sc as plsc

_C = 400
_NW = 32


def _mlp_body(x_ref, w1_ref, b1_ref, w2_ref, b2_ref, w3_ref, b3_ref, out_ref):
    h = jnp.dot(x_ref[...], w1_ref[...], preferred_element_type=jnp.float32)
    h = jnp.maximum(h + b1_ref[...], 0.0)
    d = jnp.dot(h, w2_ref[...], preferred_element_type=jnp.float32) + b2_ref[...]
    out_ref[...] = (
        jnp.dot(d, w3_ref[...], preferred_element_type=jnp.float32) + b3_ref[...]
    )


def _mlp(x, W1, b1, W2, b2, W3, b3):
    n, d = x.shape
    h = W1.shape[1]
    h2 = W2.shape[1]
    out_d = W3.shape[1]
    blk = 1000
    return pl.pallas_call(
        _mlp_body,
        grid=(n // blk,),
        in_specs=[
            pl.BlockSpec((blk, d), lambda i: (i, 0)),
            pl.BlockSpec((d, h), lambda i: (0, 0)),
            pl.BlockSpec((1, h), lambda i: (0, 0)),
            pl.BlockSpec((h, h2), lambda i: (0, 0)),
            pl.BlockSpec((1, h2), lambda i: (0, 0)),
            pl.BlockSpec((h2, out_d), lambda i: (0, 0)),
            pl.BlockSpec((1, out_d), lambda i: (0, 0)),
        ],
        out_specs=pl.BlockSpec((blk, out_d), lambda i: (i, 0)),
        out_shape=jax.ShapeDtypeStruct((n, out_d), jnp.float32),
    )(x, W1, b1.reshape(1, -1), W2, b2.reshape(1, -1), W3, b3.reshape(1, -1))


_NB = 4


def _sc_gather(x_dnn, x, idx_l, idx_r, n_chunks):
    n_pad, c = idx_l.shape
    d = x.shape[1]
    ipw = n_pad // _NW
    mesh = plsc.VectorSubcoreMesh(core_axis_name="c", subcore_axis_name="s")

    @functools.partial(
        pl.kernel,
        mesh=mesh,
        out_type=[
            jax.ShapeDtypeStruct((2, n_chunks, c, d), jnp.float32),
            jax.ShapeDtypeStruct((2, n_chunks, c, d), jnp.float32),
        ],
        scratch_types=[
            pltpu.VMEM((c,), jnp.int32),
            pltpu.VMEM((c, d), jnp.float32),
            pltpu.VMEM((c, d), jnp.float32),
            pltpu.SemaphoreType.DMA,
            pltpu.SemaphoreType.DMA,
            pltpu.SemaphoreType.DMA,
        ],
    )
    def k(dnn_hbm, x_hbm, idxl_hbm, idxr_hbm, emb_hbm, feat_hbm,
          idx_v, rows0, rows1, gsem, wsem0, wsem1):
        wid = lax.axis_index("s") * 2 + lax.axis_index("c")
        bufs = (rows0, rows1)
        wsems = (wsem0, wsem1)

        jobs = (
            (dnn_hbm, idxl_hbm, emb_hbm, 0),
            (dnn_hbm, idxr_hbm, emb_hbm, 1),
            (x_hbm, idxl_hbm, feat_hbm, 0),
            (x_hbm, idxr_hbm, feat_hbm, 1),
        )
        T = 4 * ipw

        def parts(t):
            j, i = divmod(t, ipw)
            table, idx_hbm, out_hbm, side = jobs[j]
            ch = wid + i * _NW
            live = i * _NW + _NW - 1 < n_chunks
            return table, idx_hbm, out_hbm, side, ch, live

        def w_copy(t):
            _, _, out_hbm, side, ch, _ = parts(t)
            b = t % 2
            return pltpu.make_async_copy(bufs[b], out_hbm.at[side, ch],
                                         wsems[b])

        def maybe(t, fn):
            _, _, _, _, ch, live = parts(t)
            if live:
                fn()
            else:
                pl.when(ch < n_chunks)(fn)

        for t in range(T):
            table, idx_hbm, out_hbm, side, ch, live = parts(t)
            b = t % 2
            if t >= 2:
                maybe(t - 2, w_copy(t - 2).wait)

            def step(table=table, idx_hbm=idx_hbm, ch=ch, b=b, t=t):
                pltpu.sync_copy(idx_hbm.at[ch], idx_v)
                pltpu.async_copy(table.at[idx_v], bufs[b], gsem).wait()
                w_copy(t).start()

            if live:
                step()
            else:
                pl.when(ch < n_chunks)(step)
        for t in range(T - 2, T):
            maybe(t, w_copy(t).wait)

    return k(x_dnn, x, idx_l, idx_r)


def kernel(x, edge_index, pair_idxs_left, pair_idxs_right, y, W_lin, b_lin,
           W_gat, a_src, a_dst, b_gat, W1, b1, W2, b2, W3, b3):
    p = pair_idxs_left.shape[0]
    x_dnn = _mlp(x, W1, b1, W2, b2, W3, b3)
    n_chunks = p // _C
    n_pad = ((n_chunks + _NW - 1) // _NW) * _NW
    pad = ((0, n_pad - n_chunks), (0, 0))
    idx_l = jnp.pad(pair_idxs_left.reshape(-1, _C), pad)
    idx_r = jnp.pad(pair_idxs_right.reshape(-1, _C), pad)
    emb, feat = _sc_gather(x_dnn, x, idx_l, idx_r, n_chunks)
    return (emb.reshape(2, p, -1), feat.reshape(2, p, -1), y)
